# baseline (speedup 1.0000x reference)
import functools

import jax
import jax.numpy as jnp
from jax import lax
from jax.experimental import pallas as pl
from jax.experimental.pallas import tpu as pltpu
from jax.experimental.pallas import tpu_sc as plsc

_NC = 2
_NS = 16
_NW = _NC * _NS
_CHUNK = 128


@functools.partial(jax.jit, static_argnums=(3, 4))
def _sc_sample(table2d, idx, cols, bpw, deg):
    nchunks = bpw // _CHUNK
    mesh = plsc.VectorSubcoreMesh(
        core_axis_name="c", subcore_axis_name="s", num_cores=_NC,
        num_subcores=_NS)

    @functools.partial(
        pl.kernel,
        out_type=jax.ShapeDtypeStruct((_NW, bpw, 16), jnp.int32),
        mesh=mesh,
        scratch_types=[
            pltpu.VMEM((nchunks, _CHUNK), jnp.int32),
            pltpu.VMEM((bpw, deg), jnp.int32),
            pltpu.VMEM((bpw, 16), jnp.int32),
            pltpu.VMEM((16,), jnp.int32),
            pltpu.SemaphoreType.DMA,
        ],
        compiler_params=pltpu.CompilerParams(
            needs_layout_passes=False, use_tc_tiling_on_sc=False),
    )
    def k(table_hbm, idx_hbm, cols_hbm, out_hbm, idx_v, rows_v, out_v,
          cols_v, sem):
        wid = lax.axis_index("s") * _NC + lax.axis_index("c")
        pltpu.sync_copy(cols_hbm, cols_v)
        pltpu.sync_copy(idx_hbm.at[wid], idx_v)
        copies = [
            pltpu.async_copy(
                table_hbm.at[idx_v.at[c]],
                rows_v.at[pl.ds(c * _CHUNK, _CHUNK)],
                sem,
            )
            for c in range(nchunks)
        ]
        for cp in copies:
            cp.wait()
        cols_vec = cols_v[...]

        def shuffle(b, carry):
            row = plsc.load_gather(
                rows_v, [jnp.full((16,), b, jnp.int32), cols_vec])
            out_v[b, :] = row
            return carry

        lax.fori_loop(0, bpw, shuffle, 0, unroll=8)
        pltpu.sync_copy(out_v, out_hbm.at[wid])

    out = k(table2d, idx.reshape(_NW, nchunks, _CHUNK), cols)
    return out.reshape(_NW * bpw, 16)


def kernel(adj_ins, adj_outs, g_id, ids, num_samples, ins):
    G, N, D = adj_ins.shape
    B = ids.shape[0]
    bpw = B // _NW
    perm = jax.random.permutation(jax.random.key(123), D).astype(jnp.int32)
    cols = lax.dynamic_slice(perm, (num_samples - 16,), (16,))
    idx = (ids + jnp.int32(N) * g_id).astype(jnp.int32)
    return lax.cond(
        ins != 0,
        lambda t_in, t_out: _sc_sample(t_in.reshape(G * N, D), idx, cols, bpw, D),
        lambda t_in, t_out: _sc_sample(t_out.reshape(G * N, D), idx, cols, bpw, D),
        adj_ins, adj_outs,
    )
